# baseline (speedup 1.0000x reference)
import jax
import jax.numpy as jnp
from jax.experimental import pallas as pl
from jax.experimental.pallas import tpu as pltpu

_N_TOKENS = 16384
_N_EMB = 1024
_DIM = 64
_COMMIT = 0.25
_BLK = 512
_GRID = _N_TOKENS // _BLK


def _vq_kernel(x_ref, w_ref, loss_ref, q_ref, ppl_ref, enc_ref,
               loss_acc, cnt_acc):
    i = pl.program_id(0)

    @pl.when(i == 0)
    def _init():
        loss_acc[0, 0] = 0.0
        cnt_acc[...] = jnp.zeros_like(cnt_acc)

    x = x_ref[...]
    w = w_ref[...]
    x_norm = jnp.sum(x * x, axis=1, keepdims=True)
    e_norm = jnp.sum(w * w, axis=1)
    g2 = jax.lax.dot_general(x, w + w, (((1,), (1,)), ((), ())),
                             preferred_element_type=jnp.float32)
    dist = (x_norm + e_norm[None, :]) - g2
    col = jax.lax.broadcasted_iota(jnp.int32, dist.shape, 1)
    idx = jnp.argmin(dist, axis=1).reshape(-1, 1).astype(jnp.int32)
    enc = (col == idx).astype(jnp.float32)
    enc_ref[...] = enc
    q = jax.lax.dot_general(enc, w, (((1,), (0,)), ((), ())),
                            preferred_element_type=jnp.float32)
    q_ref[...] = q
    diff = q - x
    loss_acc[0, 0] += jnp.sum(diff * diff)
    cnt_acc[...] += jnp.sum(enc, axis=0, keepdims=True)

    @pl.when(i == _GRID - 1)
    def _fin():
        loss_ref[...] = jnp.full((1, 1), _COMMIT * 0.5 / _N_TOKENS) * loss_acc[0, 0]
        avg = cnt_acc[...] / _N_TOKENS
        ent = jnp.sum(avg * jnp.log(avg + 1e-10), keepdims=True)
        ppl_ref[...] = jnp.exp(-ent).reshape(1, 1)


def kernel(inputs, weight):
    loss, quantized, ppl, encodings = pl.pallas_call(
        _vq_kernel,
        grid=(_GRID,),
        in_specs=[
            pl.BlockSpec((_BLK, _DIM), lambda i: (i, 0)),
            pl.BlockSpec((_N_EMB, _DIM), lambda i: (0, 0)),
        ],
        out_specs=[
            pl.BlockSpec((1, 1), lambda i: (0, 0)),
            pl.BlockSpec((_BLK, _DIM), lambda i: (i, 0)),
            pl.BlockSpec((1, 1), lambda i: (0, 0)),
            pl.BlockSpec((_BLK, _N_EMB), lambda i: (i, 0)),
        ],
        out_shape=[
            jax.ShapeDtypeStruct((1, 1), jnp.float32),
            jax.ShapeDtypeStruct((_N_TOKENS, _DIM), jnp.float32),
            jax.ShapeDtypeStruct((1, 1), jnp.float32),
            jax.ShapeDtypeStruct((_N_TOKENS, _N_EMB), jnp.float32),
        ],
        scratch_shapes=[
            pltpu.SMEM((1, 1), jnp.float32),
            pltpu.VMEM((1, _N_EMB), jnp.float32),
        ],
    )(inputs, weight)
    return (loss[0, 0], quantized, ppl[0, 0], encodings)


# speedup vs baseline: 1.2983x; 1.2983x over previous
import jax
import jax.numpy as jnp
from jax.experimental import pallas as pl
from jax.experimental.pallas import tpu as pltpu

_N_TOKENS = 16384
_N_EMB = 1024
_DIM = 64
_COMMIT = 0.25
_BLK = 2048
_GRID = _N_TOKENS // _BLK


def _vq_kernel(x_ref, w_ref, loss_ref, q_ref, ppl_ref, enc_ref,
               loss_acc, cnt_acc, enorm_ref, w2_ref):
    i = pl.program_id(0)

    @pl.when(i == 0)
    def _init():
        loss_acc[0, 0] = 0.0
        cnt_acc[...] = jnp.zeros_like(cnt_acc)
        w0 = w_ref[...]
        enorm_ref[...] = jnp.sum(w0 * w0, axis=1)[None, :]
        w2_ref[...] = w0 + w0

    x = x_ref[...]
    x_norm = jnp.sum(x * x, axis=1, keepdims=True)
    g2 = jax.lax.dot_general(x, w2_ref[...], (((1,), (1,)), ((), ())),
                             preferred_element_type=jnp.float32)
    dist = (x_norm + enorm_ref[...]) - g2
    col = jax.lax.broadcasted_iota(jnp.int32, dist.shape, 1)
    idx = jnp.argmin(dist, axis=1).reshape(-1, 1).astype(jnp.int32)
    enc = (col == idx).astype(jnp.float32)
    enc_ref[...] = enc
    q = jax.lax.dot_general(enc, w_ref[...], (((1,), (0,)), ((), ())),
                            preferred_element_type=jnp.float32)
    q_ref[...] = q
    diff = q - x
    loss_acc[0, 0] += jnp.sum(diff * diff)
    cnt_acc[...] += jnp.sum(enc, axis=0, keepdims=True)

    @pl.when(i == _GRID - 1)
    def _fin():
        loss_ref[...] = jnp.full((1, 1), _COMMIT * 0.5 / _N_TOKENS) * loss_acc[0, 0]
        avg = cnt_acc[...] / _N_TOKENS
        ent = jnp.sum(avg * jnp.log(avg + 1e-10), keepdims=True)
        ppl_ref[...] = jnp.exp(-ent).reshape(1, 1)


def kernel(inputs, weight):
    loss, quantized, ppl, encodings = pl.pallas_call(
        _vq_kernel,
        grid=(_GRID,),
        in_specs=[
            pl.BlockSpec((_BLK, _DIM), lambda i: (i, 0)),
            pl.BlockSpec((_N_EMB, _DIM), lambda i: (0, 0)),
        ],
        out_specs=[
            pl.BlockSpec((1, 1), lambda i: (0, 0)),
            pl.BlockSpec((_BLK, _DIM), lambda i: (i, 0)),
            pl.BlockSpec((1, 1), lambda i: (0, 0)),
            pl.BlockSpec((_BLK, _N_EMB), lambda i: (i, 0)),
        ],
        out_shape=[
            jax.ShapeDtypeStruct((1, 1), jnp.float32),
            jax.ShapeDtypeStruct((_N_TOKENS, _DIM), jnp.float32),
            jax.ShapeDtypeStruct((1, 1), jnp.float32),
            jax.ShapeDtypeStruct((_N_TOKENS, _N_EMB), jnp.float32),
        ],
        scratch_shapes=[
            pltpu.SMEM((1, 1), jnp.float32),
            pltpu.VMEM((1, _N_EMB), jnp.float32),
            pltpu.VMEM((1, _N_EMB), jnp.float32),
            pltpu.VMEM((_N_EMB, _DIM), jnp.float32),
        ],
        compiler_params=pltpu.CompilerParams(
            vmem_limit_bytes=60 * 1024 * 1024,
        ),
    )(inputs, weight)
    return (loss[0, 0], quantized, ppl[0, 0], encodings)
